# baseline (speedup 1.0000x reference)
import functools

import jax
import jax.numpy as jnp
from jax import lax
from jax.experimental import pallas as pl
from jax.experimental.pallas import tpu as pltpu
from jax.experimental.pallas import tpu_sc as plsc

P_IN = 32
B = 2
N = 100
C = 256
BN = B * N
SIZES = (48, 24, 12, 6)
LEVEL_OFF = (0, 2304, 2880, 3024)
HW_TOTAL = 3060
NPAIR = 16
LANES = 16

QMAX = 7
RMAX = QMAX * NPAIR
CCHUNKS = C // LANES

_LN2 = 0.6931471805599453


def _prep_body(qp_ref, qc_ref, woff_ref, boff_ref, sig_ref, idx_ref, w_ref):
    qp = qp_ref[...]
    qc = qc_ref[...]
    woff = woff_ref[...]
    boff = boff_ref[...]
    sig = sig_ref[...]

    offs = lax.dot_general(qc, woff, (((1,), (1,)), ((), ())),
                           preferred_element_type=jnp.float32) + boff
    x = qp[:, 0:1]
    y = qp[:, 1:2]
    z = qp[:, 2:3]
    r = qp[:, 3:4]
    dx = offs[:, 0:1]
    dy = offs[:, 1:2]
    dz = offs[:, 2:3]
    sx = x + dx * jnp.exp(_LN2 * (z - r))
    sy = y + dy * jnp.exp(_LN2 * (z + r))
    sz = z + dz

    inv2s2 = 1.0 / (2.0 * sig * sig)
    texp = [jnp.exp(-(sz - float(l)) ** 2 * inv2s2) for l in range(4)]
    zes = [jnp.exp(t) for t in texp]
    zsum = zes[0] + zes[1] + zes[2] + zes[3]
    zw = [e / zsum for e in zes]

    bidx = (lax.broadcasted_iota(jnp.int32, (BN, 1), 0) >= N).astype(jnp.int32)
    base_b = bidx * HW_TOTAL

    col = lax.broadcasted_iota(jnp.int32, (BN, NPAIR), 1)
    idx_acc = jnp.zeros((BN, NPAIR), jnp.int32)
    w_acc = jnp.zeros((BN, NPAIR), jnp.float32)
    for l in range(4):
        s = float(SIZES[l])
        si = SIZES[l]
        ix = jnp.clip(sx - 0.5, 0.0, s - 1.0)
        iy = jnp.clip(sy - 0.5, 0.0, s - 1.0)
        x0f = jnp.floor(ix)
        y0f = jnp.floor(iy)
        wx1 = ix - x0f
        wy1 = iy - y0f
        wx0 = 1.0 - wx1
        wy0 = 1.0 - wy1
        x0 = jnp.clip(x0f.astype(jnp.int32), 0, si - 1)
        x1 = jnp.clip(x0 + 1, 0, si - 1)
        y0 = jnp.clip(y0f.astype(jnp.int32), 0, si - 1)
        y1 = jnp.clip(y0 + 1, 0, si - 1)
        lbase = base_b + LEVEL_OFF[l]
        corners = ((y0, x0, wy0 * wx0), (y0, x1, wy0 * wx1),
                   (y1, x0, wy1 * wx0), (y1, x1, wy1 * wx1))
        for ci, (yy, xx, wc) in enumerate(corners):
            j = l * 4 + ci
            idx_j = lbase + yy * si + xx
            w_j = zw[l] * wc
            idx_acc = jnp.where(col == j, idx_j, idx_acc)
            w_acc = jnp.where(col == j, w_j, w_acc)

    idx_ref[...] = idx_acc
    w_ref[...] = jnp.broadcast_to(w_acc[:, :, None], (BN, NPAIR, LANES))


def _run_prep(qp, qc, woff, boff, sig, interpret=False):
    return pl.pallas_call(
        _prep_body,
        out_shape=[
            jax.ShapeDtypeStruct((BN, NPAIR), jnp.int32),
            jax.ShapeDtypeStruct((BN, NPAIR, LANES), jnp.float32),
        ],
        interpret=interpret,
    )(qp, qc, woff, boff, sig)


def _sc_body(table_hbm, idx_hbm, w_hbm, out_hbm, idx_v, w_v, rows_v, acc_v,
             sem_s, sem_g, sem_o):
    b = lax.axis_index("c")
    s = lax.axis_index("s")

    def work(nq, qloc):
        nr = nq * NPAIR
        rbase = b * (N * NPAIR) + qloc * NPAIR
        ci = pltpu.async_copy(idx_hbm.at[pl.ds(rbase, nr)],
                              idx_v.at[pl.ds(0, nr)], sem_s)
        cw = pltpu.async_copy(w_hbm.at[pl.ds(rbase, nr)],
                              w_v.at[pl.ds(0, nr)], sem_s)
        ci.wait()
        gch = []
        for k in range((nq + 1) // 2):
            n = min(2, nq - 2 * k) * NPAIR
            o = 2 * k * NPAIR
            gch.append(pltpu.async_copy(
                table_hbm.at[idx_v.at[pl.ds(o, n)]],
                rows_v.at[pl.ds(o, n)], sem_g))
        cw.wait()

        h1 = (nq + 1) // 2
        scat = []
        for q in range(nq):
            if q % 2 == 0:
                gch[q // 2].wait()

            def body(i, carry):
                row = q * NPAIR + i
                wv = w_v[row]
                out = list(carry)
                for cc in range(CCHUNKS):
                    out[cc] = out[cc] + rows_v[row, pl.ds(cc * LANES, LANES)] * wv
                return tuple(out)
            acc = lax.fori_loop(
                0, NPAIR, body,
                tuple(jnp.zeros((LANES,), jnp.float32) for _ in range(CCHUNKS)))
            for c in range(CCHUNKS):
                acc_v[q, pl.ds(c * LANES, LANES)] = acc[c]

            if q == h1 - 1:
                scat += [pltpu.async_copy(
                    acc_v.at[pl.ds(0, h1)],
                    out_hbm.at[b, pl.ds(qloc, h1), p], sem_o)
                    for p in range(P_IN)]

        h2 = nq - h1
        scat += [pltpu.async_copy(
            acc_v.at[pl.ds(h1, h2)],
            out_hbm.at[b, pl.ds(qloc + h1, h2), p], sem_o)
            for p in range(P_IN)]
        for cp in scat:
            cp.wait()

    @pl.when(s < 4)
    def _():
        work(7, s * 7)

    @pl.when(s >= 4)
    def _():
        work(6, 28 + (s - 4) * 6)


@functools.lru_cache(maxsize=1)
def _sc_gather():
    return pl.kernel(
        _sc_body,
        out_type=jax.ShapeDtypeStruct((B, N, P_IN, C), jnp.float32),
        mesh=plsc.VectorSubcoreMesh(core_axis_name="c", subcore_axis_name="s"),
        scratch_types=[
            pltpu.VMEM((RMAX,), jnp.int32),
            pltpu.VMEM((RMAX, LANES), jnp.float32),
            pltpu.VMEM((RMAX, C), jnp.float32),
            pltpu.VMEM((QMAX, C), jnp.float32),
            pltpu.SemaphoreType.DMA,
            pltpu.SemaphoreType.DMA,
            pltpu.SemaphoreType.DMA,
        ],
    )


def kernel(feat_l0, feat_l1, feat_l2, feat_l3, query_pos, query_content,
           W_off, b_off, sigma_z):
    feats = (feat_l0, feat_l1, feat_l2, feat_l3)
    table = jnp.concatenate(
        [jnp.transpose(f, (0, 2, 3, 1)).reshape(B, -1, C) for f in feats],
        axis=1).reshape(B * HW_TOTAL, C)

    qp = query_pos.reshape(BN, 4)
    qc = query_content.reshape(BN, C)
    boff = b_off.reshape(1, 3)
    sig = sigma_z.reshape(1, 1)

    idx2, wexp = _run_prep(qp, qc, W_off, boff, sig)
    idx = idx2.reshape(BN * NPAIR)
    wflat = wexp.reshape(BN * NPAIR, LANES)

    return _sc_gather()(table, idx, wflat)


# speedup vs baseline: 1.0316x; 1.0316x over previous
import functools

import jax
import jax.numpy as jnp
from jax import lax
from jax.experimental import pallas as pl
from jax.experimental.pallas import tpu as pltpu
from jax.experimental.pallas import tpu_sc as plsc

P_IN = 32
B = 2
N = 100
C = 256
BN = B * N
SIZES = (48, 24, 12, 6)
LEVEL_OFF = (0, 2304, 2880, 3024)
HW_TOTAL = 3060
NPAIR = 16
LANES = 16

QMAX = 7
RMAX = QMAX * NPAIR
CCHUNKS = C // LANES

_LN2 = 0.6931471805599453


def _prep_body(qp_ref, qc_ref, woff_ref, boff_ref, sig_ref, idx_ref, w_ref):
    qp = qp_ref[...]
    qc = qc_ref[...]
    woff = woff_ref[...]
    boff = boff_ref[...]
    sig = sig_ref[...]

    offs = lax.dot_general(qc, woff, (((1,), (1,)), ((), ())),
                           preferred_element_type=jnp.float32) + boff
    x = qp[:, 0:1]
    y = qp[:, 1:2]
    z = qp[:, 2:3]
    r = qp[:, 3:4]
    dx = offs[:, 0:1]
    dy = offs[:, 1:2]
    dz = offs[:, 2:3]
    sx = x + dx * jnp.exp(_LN2 * (z - r))
    sy = y + dy * jnp.exp(_LN2 * (z + r))
    sz = z + dz

    inv2s2 = 1.0 / (2.0 * sig * sig)
    texp = [jnp.exp(-(sz - float(l)) ** 2 * inv2s2) for l in range(4)]
    zes = [jnp.exp(t) for t in texp]
    zsum = zes[0] + zes[1] + zes[2] + zes[3]
    zw = [e / zsum for e in zes]

    bidx = (lax.broadcasted_iota(jnp.int32, (BN, 1), 0) >= N).astype(jnp.int32)
    base_b = bidx * HW_TOTAL

    col = lax.broadcasted_iota(jnp.int32, (BN, NPAIR), 1)
    idx_acc = jnp.zeros((BN, NPAIR), jnp.int32)
    w_acc = jnp.zeros((BN, NPAIR), jnp.float32)
    for l in range(4):
        s = float(SIZES[l])
        si = SIZES[l]
        ix = jnp.clip(sx - 0.5, 0.0, s - 1.0)
        iy = jnp.clip(sy - 0.5, 0.0, s - 1.0)
        x0f = jnp.floor(ix)
        y0f = jnp.floor(iy)
        wx1 = ix - x0f
        wy1 = iy - y0f
        wx0 = 1.0 - wx1
        wy0 = 1.0 - wy1
        x0 = jnp.clip(x0f.astype(jnp.int32), 0, si - 1)
        x1 = jnp.clip(x0 + 1, 0, si - 1)
        y0 = jnp.clip(y0f.astype(jnp.int32), 0, si - 1)
        y1 = jnp.clip(y0 + 1, 0, si - 1)
        lbase = base_b + LEVEL_OFF[l]
        corners = ((y0, x0, wy0 * wx0), (y0, x1, wy0 * wx1),
                   (y1, x0, wy1 * wx0), (y1, x1, wy1 * wx1))
        for ci, (yy, xx, wc) in enumerate(corners):
            j = l * 4 + ci
            idx_j = lbase + yy * si + xx
            w_j = zw[l] * wc
            idx_acc = jnp.where(col == j, idx_j, idx_acc)
            w_acc = jnp.where(col == j, w_j, w_acc)

    idx_ref[...] = idx_acc
    w_ref[...] = jnp.broadcast_to(w_acc[:, :, None], (BN, NPAIR, LANES))


def _run_prep(qp, qc, woff, boff, sig, interpret=False):
    return pl.pallas_call(
        _prep_body,
        out_shape=[
            jax.ShapeDtypeStruct((BN, NPAIR), jnp.int32),
            jax.ShapeDtypeStruct((BN, NPAIR, LANES), jnp.float32),
        ],
        interpret=interpret,
    )(qp, qc, woff, boff, sig)


def _sc_body(table_hbm, idx_hbm, w_hbm, out_hbm, idx_v, w_v, rows_v, acc_v,
             sem_s, sem_g, sem_o):
    b = lax.axis_index("c")
    s = lax.axis_index("s")

    def work(nq, qloc):
        nr = nq * NPAIR
        rbase = b * (N * NPAIR) + qloc * NPAIR
        ci = pltpu.async_copy(idx_hbm.at[pl.ds(rbase, nr)],
                              idx_v.at[pl.ds(0, nr)], sem_s)
        cw = pltpu.async_copy(w_hbm.at[pl.ds(rbase, nr)],
                              w_v.at[pl.ds(0, nr)], sem_s)
        ci.wait()
        cg = pltpu.async_copy(table_hbm.at[idx_v.at[pl.ds(0, nr)]],
                              rows_v.at[pl.ds(0, nr)], sem_g)
        cw.wait()
        cg.wait()

        h1 = (nq + 1) // 2
        scat = []
        for q in range(nq):
            def body(i, carry):
                row = q * NPAIR + i
                wv = w_v[row]
                out = list(carry)
                for cc in range(CCHUNKS):
                    out[cc] = out[cc] + rows_v[row, pl.ds(cc * LANES, LANES)] * wv
                return tuple(out)
            acc = lax.fori_loop(
                0, NPAIR, body,
                tuple(jnp.zeros((LANES,), jnp.float32) for _ in range(CCHUNKS)))
            for c in range(CCHUNKS):
                acc_v[q, pl.ds(c * LANES, LANES)] = acc[c]

            if q == h1 - 1:
                scat += [pltpu.async_copy(
                    acc_v.at[pl.ds(0, h1)],
                    out_hbm.at[b, pl.ds(qloc, h1), p], sem_o)
                    for p in range(P_IN)]

        h2 = nq - h1
        scat += [pltpu.async_copy(
            acc_v.at[pl.ds(h1, h2)],
            out_hbm.at[b, pl.ds(qloc + h1, h2), p], sem_o)
            for p in range(P_IN)]
        for cp in scat:
            cp.wait()

    @pl.when(s < 4)
    def _():
        work(7, s * 7)

    @pl.when(s >= 4)
    def _():
        work(6, 28 + (s - 4) * 6)


@functools.lru_cache(maxsize=1)
def _sc_gather():
    return pl.kernel(
        _sc_body,
        out_type=jax.ShapeDtypeStruct((B, N, P_IN, C), jnp.float32),
        mesh=plsc.VectorSubcoreMesh(core_axis_name="c", subcore_axis_name="s"),
        scratch_types=[
            pltpu.VMEM((RMAX,), jnp.int32),
            pltpu.VMEM((RMAX, LANES), jnp.float32),
            pltpu.VMEM((RMAX, C), jnp.float32),
            pltpu.VMEM((QMAX, C), jnp.float32),
            pltpu.SemaphoreType.DMA,
            pltpu.SemaphoreType.DMA,
            pltpu.SemaphoreType.DMA,
        ],
    )


def kernel(feat_l0, feat_l1, feat_l2, feat_l3, query_pos, query_content,
           W_off, b_off, sigma_z):
    feats = (feat_l0, feat_l1, feat_l2, feat_l3)
    table = jnp.concatenate(
        [jnp.transpose(f, (0, 2, 3, 1)).reshape(B, -1, C) for f in feats],
        axis=1).reshape(B * HW_TOTAL, C)

    qp = query_pos.reshape(BN, 4)
    qc = query_content.reshape(BN, C)
    boff = b_off.reshape(1, 3)
    sig = sigma_z.reshape(1, 1)

    idx2, wexp = _run_prep(qp, qc, W_off, boff, sig)
    idx = idx2.reshape(BN * NPAIR)
    wflat = wexp.reshape(BN * NPAIR, LANES)

    return _sc_gather()(table, idx, wflat)


# speedup vs baseline: 1.0356x; 1.0039x over previous
import functools

import jax
import jax.numpy as jnp
from jax import lax
from jax.experimental import pallas as pl
from jax.experimental.pallas import tpu as pltpu
from jax.experimental.pallas import tpu_sc as plsc

P_IN = 32
B = 2
N = 100
C = 256
BN = B * N
SIZES = (48, 24, 12, 6)
LEVEL_OFF = (0, 2304, 2880, 3024)
HW_TOTAL = 3060
NPAIR = 16
LANES = 16

QMAX = 7
RMAX = QMAX * NPAIR
CCHUNKS = C // LANES

_LN2 = 0.6931471805599453


def _prep_body(qp_ref, qc_ref, woff_ref, boff_ref, sig_ref, idx_ref, w_ref):
    qp = qp_ref[...]
    qc = qc_ref[...]
    woff = woff_ref[...]
    boff = boff_ref[...]
    sig = sig_ref[...]

    offs = lax.dot_general(qc, woff, (((1,), (1,)), ((), ())),
                           preferred_element_type=jnp.float32) + boff
    x = qp[:, 0:1]
    y = qp[:, 1:2]
    z = qp[:, 2:3]
    r = qp[:, 3:4]
    dx = offs[:, 0:1]
    dy = offs[:, 1:2]
    dz = offs[:, 2:3]
    sx = x + dx * jnp.exp(_LN2 * (z - r))
    sy = y + dy * jnp.exp(_LN2 * (z + r))
    sz = z + dz

    inv2s2 = 1.0 / (2.0 * sig * sig)
    texp = [jnp.exp(-(sz - float(l)) ** 2 * inv2s2) for l in range(4)]
    zes = [jnp.exp(t) for t in texp]
    zsum = zes[0] + zes[1] + zes[2] + zes[3]
    zw = [e / zsum for e in zes]

    bidx = (lax.broadcasted_iota(jnp.int32, (BN, 1), 0) >= N).astype(jnp.int32)
    base_b = bidx * HW_TOTAL

    col = lax.broadcasted_iota(jnp.int32, (BN, NPAIR), 1)
    idx_acc = jnp.zeros((BN, NPAIR), jnp.int32)
    w_acc = jnp.zeros((BN, NPAIR), jnp.float32)
    for l in range(4):
        s = float(SIZES[l])
        si = SIZES[l]
        ix = jnp.clip(sx - 0.5, 0.0, s - 1.0)
        iy = jnp.clip(sy - 0.5, 0.0, s - 1.0)
        x0f = jnp.floor(ix)
        y0f = jnp.floor(iy)
        wx1 = ix - x0f
        wy1 = iy - y0f
        wx0 = 1.0 - wx1
        wy0 = 1.0 - wy1
        x0 = jnp.clip(x0f.astype(jnp.int32), 0, si - 1)
        x1 = jnp.clip(x0 + 1, 0, si - 1)
        y0 = jnp.clip(y0f.astype(jnp.int32), 0, si - 1)
        y1 = jnp.clip(y0 + 1, 0, si - 1)
        lbase = base_b + LEVEL_OFF[l]
        corners = ((y0, x0, wy0 * wx0), (y0, x1, wy0 * wx1),
                   (y1, x0, wy1 * wx0), (y1, x1, wy1 * wx1))
        for ci, (yy, xx, wc) in enumerate(corners):
            j = l * 4 + ci
            idx_j = lbase + yy * si + xx
            w_j = zw[l] * wc
            idx_acc = jnp.where(col == j, idx_j, idx_acc)
            w_acc = jnp.where(col == j, w_j, w_acc)

    idx_ref[...] = idx_acc
    w_ref[...] = jnp.broadcast_to(w_acc[:, :, None], (BN, NPAIR, LANES))


def _run_prep(qp, qc, woff, boff, sig, interpret=False):
    return pl.pallas_call(
        _prep_body,
        out_shape=[
            jax.ShapeDtypeStruct((BN, NPAIR), jnp.int32),
            jax.ShapeDtypeStruct((BN, NPAIR, LANES), jnp.float32),
        ],
        interpret=interpret,
    )(qp, qc, woff, boff, sig)


def _sc_body(table_hbm, idx_hbm, w_hbm, out_hbm, idx_v, w_v, rows_v, acc_v,
             sem_s, sem_g, sem_o):
    b = lax.axis_index("c")
    s = lax.axis_index("s")

    def work(nq, qloc):
        nr = nq * NPAIR
        rbase = b * (N * NPAIR) + qloc * NPAIR
        ci = pltpu.async_copy(idx_hbm.at[pl.ds(rbase, nr)],
                              idx_v.at[pl.ds(0, nr)], sem_s)
        cw = pltpu.async_copy(w_hbm.at[pl.ds(rbase, nr)],
                              w_v.at[pl.ds(0, nr)], sem_s)
        ci.wait()
        cg = pltpu.async_copy(table_hbm.at[idx_v.at[pl.ds(0, nr)]],
                              rows_v.at[pl.ds(0, nr)], sem_g)
        cw.wait()
        cg.wait()

        for q in range(nq):
            def body(i, carry):
                row = q * NPAIR + i
                wv = w_v[row]
                out = list(carry)
                for cc in range(CCHUNKS):
                    out[cc] = out[cc] + rows_v[row, pl.ds(cc * LANES, LANES)] * wv
                return tuple(out)
            acc = lax.fori_loop(
                0, NPAIR, body,
                tuple(jnp.zeros((LANES,), jnp.float32) for _ in range(CCHUNKS)))
            for c in range(CCHUNKS):
                acc_v[q, pl.ds(c * LANES, LANES)] = acc[c]

        scat = [pltpu.async_copy(acc_v.at[pl.ds(0, nq)],
                                 out_hbm.at[b, pl.ds(qloc, nq), p], sem_o)
                for p in range(P_IN)]
        for cp in scat:
            cp.wait()

    @pl.when(s < 4)
    def _():
        work(7, s * 7)

    @pl.when(s >= 4)
    def _():
        work(6, 28 + (s - 4) * 6)


@functools.lru_cache(maxsize=1)
def _sc_gather():
    return pl.kernel(
        _sc_body,
        out_type=jax.ShapeDtypeStruct((B, N, P_IN, C), jnp.float32),
        mesh=plsc.VectorSubcoreMesh(core_axis_name="c", subcore_axis_name="s"),
        scratch_types=[
            pltpu.VMEM((RMAX,), jnp.int32),
            pltpu.VMEM((RMAX, LANES), jnp.float32),
            pltpu.VMEM((RMAX, C), jnp.float32),
            pltpu.VMEM((QMAX, C), jnp.float32),
            pltpu.SemaphoreType.DMA,
            pltpu.SemaphoreType.DMA,
            pltpu.SemaphoreType.DMA,
        ],
    )


def kernel(feat_l0, feat_l1, feat_l2, feat_l3, query_pos, query_content,
           W_off, b_off, sigma_z):
    feats = (feat_l0, feat_l1, feat_l2, feat_l3)
    table = jnp.concatenate(
        [jnp.transpose(f, (0, 2, 3, 1)).reshape(B, -1, C) for f in feats],
        axis=1).reshape(B * HW_TOTAL, C)

    qp = query_pos.reshape(BN, 4)
    qc = query_content.reshape(BN, C)
    boff = b_off.reshape(1, 3)
    sig = sigma_z.reshape(1, 1)

    idx2, wexp = _run_prep(qp, qc, W_off, boff, sig)
    idx = idx2.reshape(BN * NPAIR)
    wflat = wexp.reshape(BN * NPAIR, LANES)

    return _sc_gather()(table, idx, wflat)
